# baseline (speedup 1.0000x reference)
import functools

import jax
import jax.numpy as jnp
from jax.experimental import pallas as pl
from jax.experimental.pallas import tpu as pltpu

_BIG = 16384.0
_KP = 128


def _triplet_block(a_ref, f_ref, l_ref, m_ref, o_ref, ah_s, al_s, b_s, acc,
                   *, nsteps, ba):
    i = pl.program_id(0)
    n, d = f_ref.shape

    @pl.when(i == 0)
    def _():
        f = f_ref[...]
        fsq = jnp.sum(f * f, axis=1, keepdims=True)
        lab = l_ref[...]
        hi_iota = jax.lax.broadcasted_iota(jnp.int32, (n, 32), 1)
        lo_iota = jax.lax.broadcasted_iota(jnp.int32, (n, 16), 1)
        hi_m = (lab >> 4) == hi_iota
        lo_m = (lab & 15) == lo_iota
        oh_hi = jnp.where(hi_m, 1.0, 0.0)
        oh_lo = jnp.where(lo_m, 1.0, 0.0)
        bh_hi = jnp.where(hi_m, _BIG, 0.0)
        bh_lo = jnp.where(lo_m, _BIG, 0.0)
        z_hi = jnp.zeros((n, 32), jnp.float32)
        z_lo = jnp.zeros((n, 16), jnp.float32)
        ones = jnp.ones((n, 1), jnp.float32)
        pad = jnp.zeros((n, _KP - (d + 1 + 32 + 16)), jnp.float32)
        m2f = -2.0 * f
        b_s[...] = jnp.concatenate([f, fsq, oh_hi, oh_lo, pad], axis=1)
        ah_s[...] = jnp.concatenate([m2f, ones, bh_hi, z_lo, pad], axis=1)
        al_s[...] = jnp.concatenate([m2f, ones, z_hi, bh_lo, pad], axis=1)

    dims = (((1,), (1,)), ((), ()))
    row = pl.ds(i * ba, ba)
    b_all = b_s[...]
    b_a = jax.lax.dot_general(ah_s[row, :], b_all, dims,
                              preferred_element_type=jnp.float32)
    b_b = jax.lax.dot_general(al_s[row, :], b_all, dims,
                              preferred_element_type=jnp.float32)

    min_a = jnp.min(b_a, axis=1, keepdims=True)
    min_b = jnp.min(b_b, axis=1, keepdims=True)
    max_s = jnp.max(b_a + b_b, axis=1, keepdims=True)

    a = a_ref[...]
    asq = jnp.sum(a * a, axis=1, keepdims=True)

    half = 0.5 * _BIG
    cand_a = jnp.where(min_a < half, min_a, jnp.inf)
    cand_b = jnp.where(min_b < half, min_b, jnp.inf)
    neg_t = jnp.minimum(cand_a, cand_b)
    valid = neg_t < half

    pos_d2 = jnp.maximum(0.5 * (max_s - 2.0 * _BIG) + asq, 0.0)
    neg_d2 = jnp.maximum(neg_t + asq, 0.0)

    margin = m_ref[0, 0]
    per = jnp.maximum(jnp.sqrt(pos_d2) - jnp.sqrt(neg_d2) + margin, 0.0)
    per = jnp.where(valid, per, 0.0)

    s = jnp.sum(per, axis=0, keepdims=True)[0, 0]
    c = jnp.sum(valid.astype(jnp.float32), axis=0, keepdims=True)[0, 0]
    tot_s = jnp.where(i == 0, 0.0, acc[0, 0]) + s
    tot_c = jnp.where(i == 0, 0.0, acc[1, 0]) + c
    acc[0, 0] = tot_s
    acc[1, 0] = tot_c

    @pl.when(i == nsteps - 1)
    def _():
        loss = jnp.where(tot_c > 0.0, tot_s / jnp.maximum(tot_c, 1.0), 0.0)
        o_ref[...] = jnp.full((1, 1), loss, jnp.float32)


def kernel(features, labels, margin):
    n, d = features.shape
    ba = 512
    nsteps = n // ba
    labels_col = labels.reshape(n, 1).astype(jnp.int32)
    margin_arr = jnp.asarray(margin, jnp.float32).reshape(1, 1)

    out = pl.pallas_call(
        functools.partial(_triplet_block, nsteps=nsteps, ba=ba),
        grid=(nsteps,),
        in_specs=[
            pl.BlockSpec((ba, d), lambda i: (i, 0)),
            pl.BlockSpec((n, d), lambda i: (0, 0)),
            pl.BlockSpec((n, 1), lambda i: (0, 0)),
            pl.BlockSpec((1, 1), lambda i: (0, 0)),
        ],
        out_specs=pl.BlockSpec((1, 1), lambda i: (0, 0)),
        out_shape=jax.ShapeDtypeStruct((1, 1), jnp.float32),
        scratch_shapes=[pltpu.VMEM((n, _KP), jnp.float32),
                        pltpu.VMEM((n, _KP), jnp.float32),
                        pltpu.VMEM((n, _KP), jnp.float32),
                        pltpu.SMEM((2, 1), jnp.float32)],
    )(features, features, labels_col, margin_arr)
    return out[0, 0]


# speedup vs baseline: 1.3649x; 1.3649x over previous
import functools

import jax
import jax.numpy as jnp
from jax.experimental import pallas as pl
from jax.experimental.pallas import tpu as pltpu

_KP = 64
_INF = float("inf")


def _triplet_block(a_ref, f_ref, lr_ref, lc_ref, m_ref, o_ref, ah_s, b_s, acc,
                   *, nsteps, ba):
    i = pl.program_id(0)
    n, d = f_ref.shape

    @pl.when(i == 0)
    def _():
        f = f_ref[...]
        fsq = jnp.sum(f * f, axis=1, keepdims=True)
        ones = jnp.ones((n, 1), jnp.float32)
        pad = jnp.zeros((n, _KP - (d + 1)), jnp.float32)
        b_s[...] = jnp.concatenate([f, fsq, pad], axis=1)
        ah_s[...] = jnp.concatenate([-2.0 * f, ones, pad], axis=1)

    dims = (((1,), (1,)), ((), ()))
    row = pl.ds(i * ba, ba)
    t = jax.lax.dot_general(ah_s[row, :], b_s[...], dims,
                            preferred_element_type=jnp.float32)

    pos_mask = lr_ref[...] == lc_ref[...]
    pos_t = jnp.max(jnp.where(pos_mask, t, -_INF), axis=1, keepdims=True)
    neg_t = jnp.min(jnp.where(pos_mask, _INF, t), axis=1, keepdims=True)
    valid = neg_t < _INF

    a = a_ref[...]
    asq = jnp.sum(a * a, axis=1, keepdims=True)
    pos_d2 = jnp.maximum(pos_t + asq, 0.0)
    neg_d2 = jnp.maximum(jnp.where(valid, neg_t, 0.0) + asq, 0.0)

    margin = m_ref[0, 0]
    per = jnp.maximum(jnp.sqrt(pos_d2) - jnp.sqrt(neg_d2) + margin, 0.0)
    per = jnp.where(valid, per, 0.0)

    s = jnp.sum(per, axis=0, keepdims=True)[0, 0]
    c = jnp.sum(valid.astype(jnp.float32), axis=0, keepdims=True)[0, 0]
    tot_s = jnp.where(i == 0, 0.0, acc[0, 0]) + s
    tot_c = jnp.where(i == 0, 0.0, acc[1, 0]) + c
    acc[0, 0] = tot_s
    acc[1, 0] = tot_c

    @pl.when(i == nsteps - 1)
    def _():
        loss = jnp.where(tot_c > 0.0, tot_s / jnp.maximum(tot_c, 1.0), 0.0)
        o_ref[...] = jnp.full((1, 1), loss, jnp.float32)


def kernel(features, labels, margin):
    n, d = features.shape
    ba = 512
    nsteps = n // ba
    labels_col = labels.reshape(n, 1).astype(jnp.int32)
    labels_row = labels.reshape(1, n).astype(jnp.int32)
    margin_arr = jnp.asarray(margin, jnp.float32).reshape(1, 1)

    out = pl.pallas_call(
        functools.partial(_triplet_block, nsteps=nsteps, ba=ba),
        grid=(nsteps,),
        in_specs=[
            pl.BlockSpec((ba, d), lambda i: (i, 0)),
            pl.BlockSpec((n, d), lambda i: (0, 0)),
            pl.BlockSpec((ba, 1), lambda i: (i, 0)),
            pl.BlockSpec((1, n), lambda i: (0, 0)),
            pl.BlockSpec((1, 1), lambda i: (0, 0)),
        ],
        out_specs=pl.BlockSpec((1, 1), lambda i: (0, 0)),
        out_shape=jax.ShapeDtypeStruct((1, 1), jnp.float32),
        scratch_shapes=[pltpu.VMEM((n, _KP), jnp.float32),
                        pltpu.VMEM((n, _KP), jnp.float32),
                        pltpu.SMEM((2, 1), jnp.float32)],
    )(features, features, labels_col, labels_row, margin_arr)
    return out[0, 0]
